# baseline (speedup 1.0000x reference)
import functools

import jax
import jax.numpy as jnp
from jax import lax
from jax.experimental import pallas as pl
from jax.experimental.pallas import tpu as pltpu
from jax.experimental.pallas import tpu_sc as plsc

NC = 2
NS = 16
LANES = 16
NW = NC * NS

B = 16384
K = 32
M = 100000
BPW = B // NW
CHUNK = 128
NCHUNK = BPW // CHUNK
NGRP = BPW // LANES

_mesh = plsc.VectorSubcoreMesh(core_axis_name="c", subcore_axis_name="s")


@functools.partial(
    pl.kernel,
    out_type=jax.ShapeDtypeStruct((B,), jnp.float32),
    mesh=_mesh,
    scratch_types=[
        pltpu.VMEM((BPW,), jnp.int32),
        pltpu.VMEM((BPW,), jnp.int32),
        pltpu.VMEM((K * BPW,), jnp.int32),
        pltpu.VMEM((BPW, K), jnp.float32),
        pltpu.VMEM((K * BPW,), jnp.float32),
        pltpu.VMEM((BPW,), jnp.float32),
        pltpu.SemaphoreType.DMA,
        pltpu.SemaphoreType.DMA,
    ],
    compiler_params=pltpu.CompilerParams(
        needs_layout_passes=False, use_tc_tiling_on_sc=False),
)
def _policy_lr_sc(rows_hbm, cols_hbm, l_hbm, rflat_hbm, out_hbm,
                  rows_v, cols_v, ridx_v, lg_v, rgt_v, res_v, sem_l, sem_r):
    wid = lax.axis_index("s") * NC + lax.axis_index("c")
    base = wid * BPW

    pltpu.sync_copy(rows_hbm.at[pl.ds(base, BPW)], rows_v)
    pltpu.sync_copy(cols_hbm.at[pl.ds(base, BPW)], cols_v)

    l_copies = [
        pltpu.async_copy(
            l_hbm.at[rows_v.at[pl.ds(c * CHUNK, CHUNK)]],
            lg_v.at[pl.ds(c * CHUNK, CHUNK)],
            sem_l,
        )
        for c in range(NCHUNK)
    ]

    def build_idx(j, carry):
        c16 = cols_v[pl.ds(j * LANES, LANES)]
        for k in range(K):
            ridx_v[pl.ds(k * BPW + j * LANES, LANES)] = c16 + k * M
        return carry

    lax.fori_loop(0, NGRP, build_idx, 0)

    def fire_r(k, carry):
        for c in range(NCHUNK):
            off = k * BPW + c * CHUNK
            pltpu.async_copy(
                rflat_hbm.at[ridx_v.at[pl.ds(off, CHUNK)]],
                rgt_v.at[pl.ds(off, CHUNK)],
                sem_r,
            )
        return carry

    lax.fori_loop(0, K, fire_r, 0)

    for cp in l_copies:
        cp.wait()
    pltpu.make_async_copy(out_hbm, rgt_v, sem_r).wait()

    iota = lax.iota(jnp.int32, LANES)

    def dot_group(g, carry):
        row0 = g * LANES
        acc = jnp.zeros((LANES,), jnp.float32)
        for k in range(K):
            lv = plsc.load_gather(
                lg_v, [row0 + iota, jnp.full((LANES,), k, jnp.int32)])
            rv = rgt_v[pl.ds(k * BPW + row0, LANES)]
            acc = acc + lv * rv
        res_v[pl.ds(row0, LANES)] = acc
        return carry

    lax.fori_loop(0, NGRP, dot_group, 0)

    pltpu.sync_copy(res_v, out_hbm.at[pl.ds(base, BPW)])


def kernel(rows, cols, L, R, log_sigma):
    res = _policy_lr_sc(
        rows.astype(jnp.int32),
        cols.astype(jnp.int32),
        L,
        R.reshape(-1),
    )
    return res, jnp.clip(log_sigma, -2.5, 0.0)
